# baseline (speedup 1.0000x reference)
import functools

import jax
import jax.numpy as jnp
from jax import lax
from jax.experimental import pallas as pl
from jax.experimental.pallas import tpu as pltpu
from jax.experimental.pallas import tpu_sc as plsc


def _mm1_body(x_ref, w_ref, b_ref, o_ref):
    o_ref[...] = (
        jnp.dot(x_ref[...], w_ref[...], preferred_element_type=jnp.float32)
        + b_ref[...]
    )


def _mm1(x, W, b):
    n, d = x.shape
    h = W.shape[1]
    bn = 2000
    return pl.pallas_call(
        _mm1_body,
        grid=(n // bn,),
        in_specs=[
            pl.BlockSpec((bn, d), lambda i: (i, 0)),
            pl.BlockSpec((d, h), lambda i: (0, 0)),
            pl.BlockSpec((1, h), lambda i: (0, 0)),
        ],
        out_specs=pl.BlockSpec((bn, h), lambda i: (i, 0)),
        out_shape=jax.ShapeDtypeStruct((n, h), jnp.float32),
    )(x, W, b.reshape(1, h))


def _mm2_body(p0_ref, p1_ref, w_ref, b_ref, o_ref):
    hidden = jnp.maximum(p0_ref[...] + p1_ref[...], 0.0)
    o_ref[...] = (
        jnp.dot(hidden, w_ref[...], preferred_element_type=jnp.float32)
        + b_ref[...]
    )


def _mm2(p0, p1, W, b):
    n, d = p0.shape
    c = W.shape[1]
    bn = 2000
    return pl.pallas_call(
        _mm2_body,
        grid=(n // bn,),
        in_specs=[
            pl.BlockSpec((bn, d), lambda i: (i, 0)),
            pl.BlockSpec((bn, d), lambda i: (i, 0)),
            pl.BlockSpec((d, c), lambda i: (0, 0)),
            pl.BlockSpec((1, c), lambda i: (0, 0)),
        ],
        out_specs=pl.BlockSpec((bn, c), lambda i: (i, 0)),
        out_shape=jax.ShapeDtypeStruct((n, c), jnp.float32),
    )(p0, p1, W, b.reshape(1, c))


def _fin_body(nclass, p0_ref, p1_ref, o_ref):
    s = (p0_ref[...] + p1_ref[...])[:, :nclass]
    m = jnp.max(s, axis=1, keepdims=True)
    s = s - m
    o_ref[...] = s - jnp.log(jnp.sum(jnp.exp(s), axis=1, keepdims=True))


def _fin(p0, p1, nclass):
    n, cp = p0.shape
    bn = 2000
    return pl.pallas_call(
        functools.partial(_fin_body, nclass),
        grid=(n // bn,),
        in_specs=[
            pl.BlockSpec((bn, cp), lambda i: (i, 0)),
            pl.BlockSpec((bn, cp), lambda i: (i, 0)),
        ],
        out_specs=pl.BlockSpec((bn, nclass), lambda i: (i, 0)),
        out_shape=jax.ShapeDtypeStruct((n, nclass), jnp.float32),
    )(p0, p1)


_K = 80
_NBUF = 4


def _bcast_lane(vec, lane):
    idx = jnp.full((16, 1), lane, dtype=jnp.int32)
    dn = lax.GatherDimensionNumbers(
        offset_dims=(), collapsed_slice_dims=(0,), start_index_map=(0,)
    )
    return lax.gather(
        vec, idx, dn, slice_sizes=(1,),
        mode=lax.GatherScatterMode.PROMISE_IN_BOUNDS,
    )


def _spmm_sc(h, epk):
    n, dp = h.shape
    _, nw, nchunks, k_ = epk.shape
    info = plsc.get_sparse_core_info()
    nc, ns = info.num_cores, info.num_subcores
    nsub = dp // 16
    zrows = 80
    full_blocks = 8
    nsteps = (nchunks - 1) // _NBUF
    assert (nchunks - 1) % _NBUF == 0
    mesh = plsc.VectorSubcoreMesh(core_axis_name="c", subcore_axis_name="s")

    @functools.partial(
        pl.kernel,
        out_type=jax.ShapeDtypeStruct((nc, n, dp), jnp.float32),
        mesh=mesh,
        compiler_params=pltpu.CompilerParams(use_tc_tiling_on_sc=False),
        scratch_types=[
            pltpu.VMEM((_NBUF, 3, k_), jnp.int32),
            pltpu.VMEM((_NBUF, k_), jnp.int32),
            pltpu.VMEM((_NBUF, k_, dp), jnp.float32),
            pltpu.VMEM_SHARED((n, dp), jnp.float32),
            pltpu.SemaphoreType.DMA((_NBUF,)),
            pltpu.SemaphoreType.DMA((_NBUF,)),
            pltpu.SemaphoreType.DMA((_NBUF,)),
        ],
    )
    def k(h_hbm, epk_hbm, out_hbm, ebuf, sbuf, rows, accum, esem, gsem, ssem):
        cid = lax.axis_index("c")
        sid = lax.axis_index("s")
        wid = cid * ns + sid
        wb_start = sid * (zrows * full_blocks)
        nblocks = jnp.where(
            sid == ns - 1, (n - (ns - 1) * zrows * full_blocks) // zrows,
            full_blocks,
        )

        def zrow(i, carry):
            for j in range(nsub):
                rows[0, i, pl.ds(j * 16, 16)] = jnp.zeros((16,), jnp.float32)
            return carry

        lax.fori_loop(0, zrows, zrow, 0)

        def zcopy(t, carry):
            pltpu.sync_copy(
                rows.at[0], accum.at[pl.ds(wb_start + t * zrows, zrows)]
            )
            return carry

        lax.fori_loop(0, nblocks, zcopy, 0)
        plsc.subcore_barrier()

        def issue_epk(i, b):
            pltpu.async_copy(epk_hbm.at[:, wid, i], ebuf.at[b], esem.at[b])

        def wait_epk(b):
            pltpu.make_async_copy(
                epk_hbm.at[:, 0, 0], ebuf.at[b], esem.at[b]
            ).wait()

        def issue_gather(i, b):
            pltpu.async_copy(h_hbm.at[ebuf.at[b, 0]], rows.at[b], gsem.at[b])

        def wait_gather(b):
            pltpu.make_async_copy(
                h_hbm.at[ebuf.at[b, 0]], rows.at[b], gsem.at[b]
            ).wait()

        def issue_scatter(b):
            pltpu.async_copy(
                rows.at[b], accum.at[sbuf.at[b]], ssem.at[b], add=True
            )

        def drain_scatter(b):
            pltpu.make_async_copy(
                rows.at[b], accum.at[sbuf.at[0]], ssem.at[b]
            ).wait()

        def scale(b):
            rb = rows.at[b]

            def grp(g, c2):
                sl = pl.ds(g * 16, 16)
                sbuf[b, sl] = ebuf[b, 1, sl]
                av_i = ebuf[b, 2, sl]
                avec = lax.bitcast_convert_type(av_i, jnp.float32)
                for ee in range(16):
                    r = g * 16 + ee
                    bc = _bcast_lane(avec, ee)
                    for jj in range(nsub):
                        rb[r, pl.ds(jj * 16, 16)] = (
                            rb[r, pl.ds(jj * 16, 16)] * bc
                        )
                return c2

            lax.fori_loop(0, k_ // 16, grp, 0)

        issue_epk(0, 0)
        issue_epk(1, 1)
        issue_epk(2, 2)
        wait_epk(0)
        issue_gather(0, 0)
        wait_epk(1)
        issue_gather(1, 1)

        def step(stp, carry):
            for b in range(_NBUF):
                i = stp * _NBUF + b

                @pl.when(i >= 2)
                def _():
                    drain_scatter((b + _NBUF - 2) % _NBUF)

                @pl.when(i + 3 < nchunks)
                def _():
                    issue_epk(i + 3, (b + 3) % _NBUF)

                @pl.when(i + 2 < nchunks)
                def _():
                    wait_epk((b + 2) % _NBUF)
                    issue_gather(i + 2, (b + 2) % _NBUF)

                wait_gather(b)
                scale(b)
                issue_scatter(b)
            return carry

        lax.fori_loop(0, nsteps, step, 0)

        tb = (nchunks - 1) % _NBUF
        drain_scatter((tb + _NBUF - 2) % _NBUF)
        drain_scatter((tb + _NBUF - 1) % _NBUF)
        wait_gather(tb)
        scale(tb)
        issue_scatter(tb)
        drain_scatter(tb)
        plsc.subcore_barrier()

        def wb(t, carry):
            off = wb_start + t * zrows
            pltpu.sync_copy(
                accum.at[pl.ds(off, zrows)], out_hbm.at[cid, pl.ds(off, zrows)]
            )
            return carry

        lax.fori_loop(0, nblocks, wb, 0)

    return k(h, epk)


def kernel(x, edge_index, adj_values, W1, b1, W2, b2):
    n = x.shape[0]
    c = W2.shape[1]
    cp = 48
    e = adj_values.shape[0]
    nw = 32
    nchunks = e // (nw * _K)
    row3 = edge_index[0].reshape(1, nw, nchunks, _K)
    col3 = edge_index[1].reshape(1, nw, nchunks, _K)
    adj3 = lax.bitcast_convert_type(adj_values, jnp.int32).reshape(
        1, nw, nchunks, _K
    )
    epk = jnp.concatenate([col3, row3, adj3], axis=0)
    w2p = jnp.pad(W2, ((0, 0), (0, cp - c)))
    b2p = jnp.pad(b2, (0, cp - c))

    h = _mm1(x, W1, b1)
    p1 = _spmm_sc(h, epk)
    h2 = _mm2(p1[0], p1[1], w2p, b2p)
    p2 = _spmm_sc(h2, epk)
    return _fin(p2[0], p2[1], c)


# speedup vs baseline: 1.0102x; 1.0102x over previous
import functools

import jax
import jax.numpy as jnp
from jax import lax
from jax.experimental import pallas as pl
from jax.experimental.pallas import tpu as pltpu
from jax.experimental.pallas import tpu_sc as plsc


def _mm1_body(x_ref, w_ref, b_ref, o_ref):
    o_ref[...] = (
        jnp.dot(x_ref[...], w_ref[...], preferred_element_type=jnp.float32)
        + b_ref[...]
    )


def _mm1(x, W, b):
    n, d = x.shape
    h = W.shape[1]
    bn = 2000
    return pl.pallas_call(
        _mm1_body,
        grid=(n // bn,),
        in_specs=[
            pl.BlockSpec((bn, d), lambda i: (i, 0)),
            pl.BlockSpec((d, h), lambda i: (0, 0)),
            pl.BlockSpec((1, h), lambda i: (0, 0)),
        ],
        out_specs=pl.BlockSpec((bn, h), lambda i: (i, 0)),
        out_shape=jax.ShapeDtypeStruct((n, h), jnp.float32),
    )(x, W, b.reshape(1, h))


def _mm2_body(p0_ref, p1_ref, w_ref, b_ref, o_ref):
    hidden = jnp.maximum(p0_ref[...] + p1_ref[...], 0.0)
    o_ref[...] = (
        jnp.dot(hidden, w_ref[...], preferred_element_type=jnp.float32)
        + b_ref[...]
    )


def _mm2(p0, p1, W, b):
    n, d = p0.shape
    c = W.shape[1]
    bn = 2000
    return pl.pallas_call(
        _mm2_body,
        grid=(n // bn,),
        in_specs=[
            pl.BlockSpec((bn, d), lambda i: (i, 0)),
            pl.BlockSpec((bn, d), lambda i: (i, 0)),
            pl.BlockSpec((d, c), lambda i: (0, 0)),
            pl.BlockSpec((1, c), lambda i: (0, 0)),
        ],
        out_specs=pl.BlockSpec((bn, c), lambda i: (i, 0)),
        out_shape=jax.ShapeDtypeStruct((n, c), jnp.float32),
    )(p0, p1, W, b.reshape(1, c))


def _fin_body(nclass, p0_ref, p1_ref, o_ref):
    s = (p0_ref[...] + p1_ref[...])[:, :nclass]
    m = jnp.max(s, axis=1, keepdims=True)
    s = s - m
    o_ref[...] = s - jnp.log(jnp.sum(jnp.exp(s), axis=1, keepdims=True))


def _fin(p0, p1, nclass):
    n, cp = p0.shape
    bn = 2000
    return pl.pallas_call(
        functools.partial(_fin_body, nclass),
        grid=(n // bn,),
        in_specs=[
            pl.BlockSpec((bn, cp), lambda i: (i, 0)),
            pl.BlockSpec((bn, cp), lambda i: (i, 0)),
        ],
        out_specs=pl.BlockSpec((bn, nclass), lambda i: (i, 0)),
        out_shape=jax.ShapeDtypeStruct((n, nclass), jnp.float32),
    )(p0, p1)


_K = 80
_NBUF = 4


def _bcast_lane(vec, lane):
    idx = jnp.full((16, 1), lane, dtype=jnp.int32)
    dn = lax.GatherDimensionNumbers(
        offset_dims=(), collapsed_slice_dims=(0,), start_index_map=(0,)
    )
    return lax.gather(
        vec, idx, dn, slice_sizes=(1,),
        mode=lax.GatherScatterMode.PROMISE_IN_BOUNDS,
    )


def _spmm_sc(h, epk, tc_tiling=True):
    n, dp = h.shape
    _, nw, nchunks, k_ = epk.shape
    info = plsc.get_sparse_core_info()
    nc, ns = info.num_cores, info.num_subcores
    nsub = dp // 16
    zrows = 80
    full_blocks = 8
    nsteps = (nchunks - 1) // _NBUF
    assert (nchunks - 1) % _NBUF == 0
    mesh = plsc.VectorSubcoreMesh(core_axis_name="c", subcore_axis_name="s")

    @functools.partial(
        pl.kernel,
        out_type=jax.ShapeDtypeStruct((nc, n, dp), jnp.float32),
        mesh=mesh,
        compiler_params=pltpu.CompilerParams(use_tc_tiling_on_sc=tc_tiling),
        scratch_types=[
            pltpu.VMEM((_NBUF, 3, k_), jnp.int32),
            pltpu.VMEM((_NBUF, k_), jnp.int32),
            pltpu.VMEM((_NBUF, k_, dp), jnp.float32),
            pltpu.VMEM_SHARED((n, dp), jnp.float32),
            pltpu.SemaphoreType.DMA((_NBUF,)),
            pltpu.SemaphoreType.DMA((_NBUF,)),
            pltpu.SemaphoreType.DMA((_NBUF,)),
        ],
    )
    def k(h_hbm, epk_hbm, out_hbm, ebuf, sbuf, rows, accum, esem, gsem, ssem):
        cid = lax.axis_index("c")
        sid = lax.axis_index("s")
        wid = cid * ns + sid
        wb_start = sid * (zrows * full_blocks)
        nblocks = jnp.where(
            sid == ns - 1, (n - (ns - 1) * zrows * full_blocks) // zrows,
            full_blocks,
        )

        def zrow(i, carry):
            for j in range(nsub):
                rows[0, i, pl.ds(j * 16, 16)] = jnp.zeros((16,), jnp.float32)
            return carry

        lax.fori_loop(0, zrows, zrow, 0)

        def zcopy(t, carry):
            pltpu.sync_copy(
                rows.at[0], accum.at[pl.ds(wb_start + t * zrows, zrows)]
            )
            return carry

        lax.fori_loop(0, nblocks, zcopy, 0)
        plsc.subcore_barrier()

        def issue_epk(i, b):
            pltpu.async_copy(epk_hbm.at[:, wid, i], ebuf.at[b], esem.at[b])

        def wait_epk(b):
            pltpu.make_async_copy(
                epk_hbm.at[:, 0, 0], ebuf.at[b], esem.at[b]
            ).wait()

        def issue_gather(i, b):
            pltpu.async_copy(h_hbm.at[ebuf.at[b, 0]], rows.at[b], gsem.at[b])

        def wait_gather(b):
            pltpu.make_async_copy(
                h_hbm.at[ebuf.at[b, 0]], rows.at[b], gsem.at[b]
            ).wait()

        def issue_scatter(b):
            pltpu.async_copy(
                rows.at[b], accum.at[sbuf.at[b]], ssem.at[b], add=True
            )

        def drain_scatter(b):
            pltpu.make_async_copy(
                rows.at[b], accum.at[sbuf.at[0]], ssem.at[b]
            ).wait()

        def scale(b):
            rb = rows.at[b]

            def grp(g, c2):
                sl = pl.ds(g * 16, 16)
                sbuf[b, sl] = ebuf[b, 1, sl]
                av_i = ebuf[b, 2, sl]
                avec = lax.bitcast_convert_type(av_i, jnp.float32)
                for ee in range(16):
                    r = g * 16 + ee
                    bc = _bcast_lane(avec, ee)
                    for jj in range(nsub):
                        rb[r, pl.ds(jj * 16, 16)] = (
                            rb[r, pl.ds(jj * 16, 16)] * bc
                        )
                return c2

            lax.fori_loop(0, k_ // 16, grp, 0)

        issue_epk(0, 0)
        issue_epk(1, 1)
        issue_epk(2, 2)
        wait_epk(0)
        issue_gather(0, 0)
        wait_epk(1)
        issue_gather(1, 1)

        def step(stp, carry):
            for b in range(_NBUF):
                i = stp * _NBUF + b

                @pl.when(i >= 2)
                def _():
                    drain_scatter((b + _NBUF - 2) % _NBUF)

                @pl.when(i + 3 < nchunks)
                def _():
                    issue_epk(i + 3, (b + 3) % _NBUF)

                @pl.when(i + 2 < nchunks)
                def _():
                    wait_epk((b + 2) % _NBUF)
                    issue_gather(i + 2, (b + 2) % _NBUF)

                wait_gather(b)
                scale(b)
                issue_scatter(b)
            return carry

        lax.fori_loop(0, nsteps, step, 0)

        tb = (nchunks - 1) % _NBUF
        drain_scatter((tb + _NBUF - 2) % _NBUF)
        drain_scatter((tb + _NBUF - 1) % _NBUF)
        wait_gather(tb)
        scale(tb)
        issue_scatter(tb)
        drain_scatter(tb)
        plsc.subcore_barrier()

        def wb(t, carry):
            off = wb_start + t * zrows
            pltpu.sync_copy(
                accum.at[pl.ds(off, zrows)], out_hbm.at[cid, pl.ds(off, zrows)]
            )
            return carry

        lax.fori_loop(0, nblocks, wb, 0)

    return k(h, epk)


def kernel(x, edge_index, adj_values, W1, b1, W2, b2):
    n = x.shape[0]
    c = W2.shape[1]
    cp = 48
    e = adj_values.shape[0]
    nw = 32
    nchunks = e // (nw * _K)
    row3 = edge_index[0].reshape(1, nw, nchunks, _K)
    col3 = edge_index[1].reshape(1, nw, nchunks, _K)
    adj3 = lax.bitcast_convert_type(adj_values, jnp.int32).reshape(
        1, nw, nchunks, _K
    )
    epk = jnp.concatenate([col3, row3, adj3], axis=0)
    w2p = jnp.pad(W2, ((0, 0), (0, cp - c)))
    b2p = jnp.pad(b2, (0, cp - c))

    h = _mm1(x, W1, b1)
    p1 = _spmm_sc(h, epk)
    h2 = _mm2(p1[0], p1[1], w2p, b2p)
    p2 = _spmm_sc(h2, epk, tc_tiling=False)
    return _fin(p2[0], p2[1], c)


# speedup vs baseline: 1.1504x; 1.1388x over previous
import functools

import jax
import jax.numpy as jnp
from jax import lax
from jax.experimental import pallas as pl
from jax.experimental.pallas import tpu as pltpu
from jax.experimental.pallas import tpu_sc as plsc


def _mm1_body(x_ref, w_ref, b_ref, o_ref):
    o_ref[...] = (
        jnp.dot(x_ref[...], w_ref[...], preferred_element_type=jnp.float32)
        + b_ref[...]
    )


def _mm1(x, W, b):
    n, d = x.shape
    h = W.shape[1]
    bn = 2000
    return pl.pallas_call(
        _mm1_body,
        grid=(n // bn,),
        in_specs=[
            pl.BlockSpec((bn, d), lambda i: (i, 0)),
            pl.BlockSpec((d, h), lambda i: (0, 0)),
            pl.BlockSpec((1, h), lambda i: (0, 0)),
        ],
        out_specs=pl.BlockSpec((bn, h), lambda i: (i, 0)),
        out_shape=jax.ShapeDtypeStruct((n, h), jnp.float32),
    )(x, W, b.reshape(1, h))


def _mm2_body(p0_ref, p1_ref, w_ref, b_ref, o_ref):
    hidden = jnp.maximum(p0_ref[...] + p1_ref[...], 0.0)
    o_ref[...] = (
        jnp.dot(hidden, w_ref[...], preferred_element_type=jnp.float32)
        + b_ref[...]
    )


def _mm2(p0, p1, W, b):
    n, d = p0.shape
    c = W.shape[1]
    bn = 2000
    return pl.pallas_call(
        _mm2_body,
        grid=(n // bn,),
        in_specs=[
            pl.BlockSpec((bn, d), lambda i: (i, 0)),
            pl.BlockSpec((bn, d), lambda i: (i, 0)),
            pl.BlockSpec((d, c), lambda i: (0, 0)),
            pl.BlockSpec((1, c), lambda i: (0, 0)),
        ],
        out_specs=pl.BlockSpec((bn, c), lambda i: (i, 0)),
        out_shape=jax.ShapeDtypeStruct((n, c), jnp.float32),
    )(p0, p1, W, b.reshape(1, c))


def _fin_body(nclass, p0_ref, p1_ref, o_ref):
    s = (p0_ref[...] + p1_ref[...])[:, :nclass]
    m = jnp.max(s, axis=1, keepdims=True)
    s = s - m
    o_ref[...] = s - jnp.log(jnp.sum(jnp.exp(s), axis=1, keepdims=True))


def _fin(p0, p1, nclass):
    n, cp = p0.shape
    bn = 2000
    return pl.pallas_call(
        functools.partial(_fin_body, nclass),
        grid=(n // bn,),
        in_specs=[
            pl.BlockSpec((bn, cp), lambda i: (i, 0)),
            pl.BlockSpec((bn, cp), lambda i: (i, 0)),
        ],
        out_specs=pl.BlockSpec((bn, nclass), lambda i: (i, 0)),
        out_shape=jax.ShapeDtypeStruct((n, nclass), jnp.float32),
    )(p0, p1)


_K = 80
_NCHUNKS = 125
_NBUF = 4


def _bcast_lane(vec, lane):
    idx = jnp.full((16, 1), lane, dtype=jnp.int32)
    dn = lax.GatherDimensionNumbers(
        offset_dims=(), collapsed_slice_dims=(0,), start_index_map=(0,)
    )
    return lax.gather(
        vec, idx, dn, slice_sizes=(1,),
        mode=lax.GatherScatterMode.PROMISE_IN_BOUNDS,
    )


def _spmm_sc(h, epk, tc_tiling=True):
    n, dp = h.shape
    _, nw, nchunks, k_ = epk.shape
    info = plsc.get_sparse_core_info()
    nc, ns = info.num_cores, info.num_subcores
    nsub = dp // 16
    zrows = 80
    full_blocks = 8
    nsteps = (nchunks - 1) // _NBUF
    assert (nchunks - 1) % _NBUF == 0
    mesh = plsc.VectorSubcoreMesh(core_axis_name="c", subcore_axis_name="s")

    @functools.partial(
        pl.kernel,
        out_type=jax.ShapeDtypeStruct((nc, n, dp), jnp.float32),
        mesh=mesh,
        compiler_params=pltpu.CompilerParams(use_tc_tiling_on_sc=tc_tiling),
        scratch_types=[
            pltpu.VMEM((_NBUF, 3, k_), jnp.int32),
            pltpu.VMEM((_NBUF, k_), jnp.int32),
            pltpu.VMEM((_NBUF, k_, dp), jnp.float32),
            pltpu.VMEM_SHARED((n, dp), jnp.float32),
            pltpu.SemaphoreType.DMA((_NBUF,)),
            pltpu.SemaphoreType.DMA((_NBUF,)),
            pltpu.SemaphoreType.DMA((_NBUF,)),
        ],
    )
    def k(h_hbm, epk_hbm, out_hbm, ebuf, sbuf, rows, accum, esem, gsem, ssem):
        cid = lax.axis_index("c")
        sid = lax.axis_index("s")
        wid = cid * ns + sid
        wb_start = sid * (zrows * full_blocks)
        nblocks = jnp.where(
            sid == ns - 1, (n - (ns - 1) * zrows * full_blocks) // zrows,
            full_blocks,
        )

        def zrow(i, carry):
            for j in range(nsub):
                rows[0, i, pl.ds(j * 16, 16)] = jnp.zeros((16,), jnp.float32)
            return carry

        lax.fori_loop(0, zrows, zrow, 0)

        def zcopy(t, carry):
            pltpu.sync_copy(
                rows.at[0], accum.at[pl.ds(wb_start + t * zrows, zrows)]
            )
            return carry

        lax.fori_loop(0, nblocks, zcopy, 0)
        plsc.subcore_barrier()

        def issue_epk(i, b):
            pltpu.async_copy(epk_hbm.at[:, wid, i], ebuf.at[b], esem.at[b])

        def wait_epk(b):
            pltpu.make_async_copy(
                epk_hbm.at[:, 0, 0], ebuf.at[b], esem.at[b]
            ).wait()

        def issue_gather(i, b):
            pltpu.async_copy(h_hbm.at[ebuf.at[b, 0]], rows.at[b], gsem.at[b])

        def wait_gather(b):
            pltpu.make_async_copy(
                h_hbm.at[ebuf.at[b, 0]], rows.at[b], gsem.at[b]
            ).wait()

        def issue_scatter(b):
            pltpu.async_copy(
                rows.at[b], accum.at[sbuf.at[b]], ssem.at[b], add=True
            )

        def drain_scatter(b):
            pltpu.make_async_copy(
                rows.at[b], accum.at[sbuf.at[0]], ssem.at[b]
            ).wait()

        def scale(b):
            rb = rows.at[b]

            def grp(g, c2):
                sl = pl.ds(g * 16, 16)
                sbuf[b, sl] = ebuf[b, 1, sl]
                av_i = ebuf[b, 2, sl]
                avec = lax.bitcast_convert_type(av_i, jnp.float32)
                for ee in range(16):
                    r = g * 16 + ee
                    bc = _bcast_lane(avec, ee)
                    for jj in range(nsub):
                        rb[r, pl.ds(jj * 16, 16)] = (
                            rb[r, pl.ds(jj * 16, 16)] * bc
                        )
                return c2

            lax.fori_loop(0, k_ // 16, grp, 0)

        issue_epk(0, 0)
        issue_epk(1, 1)
        issue_epk(2, 2)
        wait_epk(0)
        issue_gather(0, 0)
        wait_epk(1)
        issue_gather(1, 1)

        def step(stp, carry):
            for b in range(_NBUF):
                i = stp * _NBUF + b

                @pl.when(i >= 2)
                def _():
                    drain_scatter((b + _NBUF - 2) % _NBUF)

                @pl.when(i + 3 < nchunks)
                def _():
                    issue_epk(i + 3, (b + 3) % _NBUF)

                @pl.when(i + 2 < nchunks)
                def _():
                    wait_epk((b + 2) % _NBUF)
                    issue_gather(i + 2, (b + 2) % _NBUF)

                wait_gather(b)
                scale(b)
                issue_scatter(b)
            return carry

        lax.fori_loop(0, nsteps, step, 0)

        tb = (nchunks - 1) % _NBUF
        drain_scatter((tb + _NBUF - 2) % _NBUF)
        drain_scatter((tb + _NBUF - 1) % _NBUF)
        wait_gather(tb)
        scale(tb)
        issue_scatter(tb)
        drain_scatter(tb)
        plsc.subcore_barrier()

        def wb(t, carry):
            off = wb_start + t * zrows
            pltpu.sync_copy(
                accum.at[pl.ds(off, zrows)], out_hbm.at[cid, pl.ds(off, zrows)]
            )
            return carry

        lax.fori_loop(0, nblocks, wb, 0)

    return k(h, epk)


def kernel(x, edge_index, adj_values, W1, b1, W2, b2):
    n = x.shape[0]
    c = W2.shape[1]
    cp = 128
    e = adj_values.shape[0]
    nw = 32
    epw = e // nw
    pad = _NCHUNKS * _K - epw

    def _plane(a):
        return jnp.pad(a.reshape(nw, epw), ((0, 0), (0, pad))).reshape(
            1, nw, _NCHUNKS, _K
        )

    row3 = _plane(edge_index[0])
    col3 = _plane(edge_index[1])
    adj3 = _plane(lax.bitcast_convert_type(adj_values, jnp.int32))
    epk = jnp.concatenate([col3, row3, adj3], axis=0)
    w2p = jnp.pad(W2, ((0, 0), (0, cp - c)))
    b2p = jnp.pad(b2, (0, cp - c))

    h = _mm1(x, W1, b1)
    p1 = _spmm_sc(h, epk)
    h2 = _mm2(p1[0], p1[1], w2p, b2p)
    p2 = _spmm_sc(h2, epk)
    return _fin(p2[0], p2[1], c)
